# baseline (speedup 1.0000x reference)
import functools

import jax
import jax.numpy as jnp
import numpy as np
from jax import lax
from jax.experimental import pallas as pl
from jax.experimental.pallas import tpu as pltpu
from jax.experimental.pallas import tpu_sc as plsc

VOCAB = 1000000
EMBED_DIM = 64
NOISE_ALPHA = 5.0


_NC, _NS = 2, 16
_NW = _NC * _NS
_C0, _C1 = 128, 72


def _sc_gather_body(batch, seq, table_hbm, xlin_hbm, out_hbm,
                    idx_a, idx_b, rows_v, sem):
    wid = lax.axis_index("s") * _NC + lax.axis_index("c")
    xrows_per_w = batch // _NW
    r0 = wid * xrows_per_w

    def chunk(j, carry):
        r = r0 + j
        pltpu.sync_copy(xlin_hbm.at[2 * r], idx_a)
        pltpu.sync_copy(xlin_hbm.at[2 * r + 1, pl.ds(0, _C1)], idx_b)
        pltpu.async_copy(table_hbm.at[idx_a], rows_v.at[pl.ds(0, _C0)],
                         sem).wait()
        pltpu.async_copy(table_hbm.at[idx_b], rows_v.at[pl.ds(_C0, _C1)],
                         sem).wait()
        pltpu.sync_copy(rows_v, out_hbm.at[pl.ds(r * seq, seq)])
        return carry

    lax.fori_loop(0, xrows_per_w, chunk, 0, unroll=False)


def _sc_gather(table, xlin, batch, seq):
    nrows = batch * seq
    mesh = plsc.VectorSubcoreMesh(core_axis_name="c", subcore_axis_name="s")
    return pl.kernel(
        functools.partial(_sc_gather_body, batch, seq),
        out_type=jax.ShapeDtypeStruct((nrows, EMBED_DIM), jnp.float32),
        mesh=mesh,
        compiler_params=pltpu.CompilerParams(use_tc_tiling_on_sc=False),
        scratch_types=[
            pltpu.VMEM((_C0,), jnp.int32),
            pltpu.VMEM((_C1,), jnp.int32),
            pltpu.VMEM((seq, EMBED_DIM), jnp.float32),
            pltpu.SemaphoreType.DMA,
        ],
    )(table, xlin)


_ROT_A = (13, 15, 26, 6)
_ROT_B = (17, 29, 16, 24)


def _threefry_noise(lo, mag):
    u32 = jnp.uint32
    ks0 = u32(0)
    ks1 = u32(42)
    ks2 = ks0 ^ ks1 ^ u32(0x1BD11BDA)
    ks = (ks0, ks1, ks2)
    x0 = jnp.zeros_like(lo)
    x1 = lo + ks1

    def rotl(v, d):
        return (v << u32(d)) | (v >> u32(32 - d))

    for i in range(5):
        rots = _ROT_A if i % 2 == 0 else _ROT_B
        for r in rots:
            x0 = x0 + x1
            x1 = rotl(x1, r)
            x1 = x1 ^ x0
        x0 = x0 + ks[(i + 1) % 3]
        x1 = x1 + ks[(i + 2) % 3] + u32(i + 1)

    bits = x0 ^ x1
    fl = lax.bitcast_convert_type((bits >> u32(9)) | u32(0x3F800000),
                                  jnp.float32) - jnp.float32(1.0)
    return jnp.maximum(jnp.float32(-mag),
                       fl * jnp.float32(2.0 * mag) + jnp.float32(-mag))


def _noise_body(mag, blk_lines, blk_b, l, d, g_ref, out_ref):
    u32 = jnp.uint32
    shape = (blk_lines, 128)
    base = (pl.program_id(0) * blk_lines * 128).astype(u32)
    row = lax.broadcasted_iota(u32, shape, 0)
    col = lax.broadcasted_iota(u32, shape, 1)
    lo = base + row * u32(128) + col
    noised = g_ref[...] + _threefry_noise(lo, mag)
    a = noised[:, :d]
    b = noised[:, d:]
    out_ref[...] = jnp.concatenate(
        [a[:, None, :], b[:, None, :]], axis=1).reshape(blk_b, l, d)


def _tc_noise_add(g, mag, b, l):
    n, cols = g.shape
    blk_b = 8
    blk_lines = blk_b * l * EMBED_DIM // 128
    grid = b // blk_b
    return pl.pallas_call(
        functools.partial(_noise_body, mag, blk_lines, blk_b, l, EMBED_DIM),
        out_shape=jax.ShapeDtypeStruct((b, l, EMBED_DIM), jnp.float32),
        grid=(grid,),
        in_specs=[pl.BlockSpec((blk_lines, cols), lambda i: (i, 0))],
        out_specs=pl.BlockSpec((blk_b, l, EMBED_DIM), lambda i: (i, 0, 0)),
    )(g)


def kernel(x, table):
    b, l = x.shape
    nrows = b * l
    xlin = jnp.pad(x, ((0, 0), (0, 256 - l))).reshape(b * 2, 128)
    gathered = _sc_gather(table, xlin, b, l)

    dims = np.float32(l * EMBED_DIM)
    mag = np.float32(NOISE_ALPHA) / np.sqrt(dims)

    flat = gathered.reshape(nrows * EMBED_DIM // 128, 128)
    return _tc_noise_add(flat, mag, b, l)


# speedup vs baseline: 1.5134x; 1.5134x over previous
import functools

import jax
import jax.numpy as jnp
import numpy as np
from jax import lax
from jax.experimental import pallas as pl
from jax.experimental.pallas import tpu as pltpu
from jax.experimental.pallas import tpu_sc as plsc

VOCAB = 1000000
EMBED_DIM = 64
NOISE_ALPHA = 5.0


_NC, _NS = 2, 16
_NW = _NC * _NS


def _sc_gather_body(batch, seq, table_hbm, x1_hbm, x2_hbm, out_hbm,
                    iv0, iv1, iv2, rv0, rv1, rv2, si0, si1, si2,
                    sg0, sg1, sg2, ss0, ss1, ss2):
    wid = lax.axis_index("s") * _NC + lax.axis_index("c")
    xrows_per_w = batch // _NW
    r0 = wid * xrows_per_w
    n16 = seq // 16
    tail = seq - n16 * 16
    idx_v = (iv0, iv1, iv2)
    rows_v = (rv0, rv1, rv2)
    sem_i = (si0, si1, si2)
    sem_g = (sg0, sg1, sg2)
    sem_s = (ss0, ss1, ss2)

    def issue_idx(r, b):
        pltpu.async_copy(x1_hbm.at[r], idx_v[b].at[pl.ds(0, 128)], sem_i[b])
        pltpu.async_copy(x2_hbm.at[r], idx_v[b].at[pl.ds(128, 128)], sem_i[b])

    def wait_idx(b):
        pltpu.make_async_copy(x1_hbm.at[0],
                              idx_v[b].at[pl.ds(0, 128)], sem_i[b]).wait()
        pltpu.make_async_copy(x2_hbm.at[0],
                              idx_v[b].at[pl.ds(128, 128)], sem_i[b]).wait()

    def fire(b):
        def fire16(k, c):
            v = idx_v[b][pl.ds(k * 16, 16)]
            for t in range(16):
                pltpu.async_copy(table_hbm.at[pl.ds(v[t], 1)],
                                 rows_v[b].at[pl.ds(k * 16 + t, 1)],
                                 sem_g[b])
            return c

        lax.fori_loop(0, n16, fire16, 0, unroll=False)
        if tail:
            v = idx_v[b][pl.ds(n16 * 16, 16)]
            for t in range(tail):
                pltpu.async_copy(table_hbm.at[pl.ds(v[t], 1)],
                                 rows_v[b].at[pl.ds(n16 * 16 + t, 1)],
                                 sem_g[b])

    def row_step(r, b, do_prefetch):
        bp = (b + 1) % 3
        bn = (b + 2) % 3
        if do_prefetch:
            @pl.when(r + 2 < r0 + xrows_per_w)
            def _():
                issue_idx(r + 2, bn)

        wait_idx(b)

        @pl.when(r >= r0 + 3)
        def _():
            pltpu.make_async_copy(table_hbm.at[pl.ds(0, seq)],
                                  rows_v[b], sem_s[b]).wait()

        fire(b)

        @pl.when(r >= r0 + 2)
        def _():
            pltpu.make_async_copy(table_hbm.at[pl.ds(0, seq)],
                                  rows_v[bp], sem_g[bp]).wait()
            pltpu.async_copy(rows_v[bp],
                             out_hbm.at[pl.ds((r - 2) * seq, seq)],
                             sem_s[bp])

    issue_idx(r0, 0)
    issue_idx(r0 + 1, 1)

    nfull = xrows_per_w // 3

    def step(j, carry):
        for b in (0, 1, 2):
            row_step(r0 + 3 * j + b, b, True)
        return carry

    lax.fori_loop(0, nfull, step, 0, unroll=False)

    for k in range(3 * nfull, xrows_per_w):
        row_step(r0 + k, k % 3, False)

    for k in (xrows_per_w - 2, xrows_per_w - 1):
        b = k % 3
        pltpu.make_async_copy(table_hbm.at[pl.ds(0, seq)], rows_v[b],
                              sem_g[b]).wait()
        pltpu.sync_copy(rows_v[b],
                        out_hbm.at[pl.ds((r0 + k) * seq, seq)])
    b3 = (xrows_per_w - 3) % 3
    pltpu.make_async_copy(table_hbm.at[pl.ds(0, seq)], rows_v[b3],
                          sem_s[b3]).wait()


def _sc_gather(table, x1, x2, batch, seq):
    nrows = batch * seq
    mesh = plsc.VectorSubcoreMesh(core_axis_name="c", subcore_axis_name="s")
    return pl.kernel(
        functools.partial(_sc_gather_body, batch, seq),
        out_type=jax.ShapeDtypeStruct((nrows, EMBED_DIM), jnp.float32),
        mesh=mesh,
        compiler_params=pltpu.CompilerParams(use_tc_tiling_on_sc=True),
        scratch_types=(
            [pltpu.VMEM((256,), jnp.int32)] * 3
            + [pltpu.VMEM((seq, EMBED_DIM), jnp.float32)] * 3
            + [pltpu.SemaphoreType.DMA] * 9
        ),
    )(table, x1, x2)


_ROT_A = (13, 15, 26, 6)
_ROT_B = (17, 29, 16, 24)


def _threefry_noise(lo, mag):
    u32 = jnp.uint32
    ks0 = u32(0)
    ks1 = u32(42)
    ks2 = ks0 ^ ks1 ^ u32(0x1BD11BDA)
    ks = (ks0, ks1, ks2)
    x0 = jnp.zeros_like(lo)
    x1 = lo + ks1

    def rotl(v, d):
        return (v << u32(d)) | (v >> u32(32 - d))

    for i in range(5):
        rots = _ROT_A if i % 2 == 0 else _ROT_B
        for r in rots:
            x0 = x0 + x1
            x1 = rotl(x1, r)
            x1 = x1 ^ x0
        x0 = x0 + ks[(i + 1) % 3]
        x1 = x1 + ks[(i + 2) % 3] + u32(i + 1)

    bits = x0 ^ x1
    fl = lax.bitcast_convert_type((bits >> u32(9)) | u32(0x3F800000),
                                  jnp.float32) - jnp.float32(1.0)
    return jnp.maximum(jnp.float32(-mag),
                       fl * jnp.float32(2.0 * mag) + jnp.float32(-mag))


def _noise_body(mag, blk_l, d, blk_b, l, g_ref, out_ref):
    u32 = jnp.uint32
    shape = (blk_l, d, blk_b)
    i = pl.program_id(0)
    j = pl.program_id(1)
    li = lax.broadcasted_iota(u32, shape, 0) + (i * blk_l).astype(u32)
    di = lax.broadcasted_iota(u32, shape, 1)
    bi = lax.broadcasted_iota(u32, shape, 2) + (j * blk_b).astype(u32)
    lo = bi * u32(l * d) + li * u32(d) + di
    out_ref[...] = g_ref[...] + _threefry_noise(lo, mag)


def _tc_noise_add(g_t, mag, b, l):
    blk_l, blk_b = 25, 512
    grid = (l // blk_l, b // blk_b)
    spec = pl.BlockSpec((blk_l, EMBED_DIM, blk_b), lambda i, j: (i, 0, j))
    return pl.pallas_call(
        functools.partial(_noise_body, mag, blk_l, EMBED_DIM, blk_b, l),
        out_shape=jax.ShapeDtypeStruct((l, EMBED_DIM, b), jnp.float32),
        grid=grid,
        in_specs=[spec],
        out_specs=spec,
    )(g_t)


def kernel(x, table):
    b, l = x.shape
    x1 = x[:, :128]
    x2 = jnp.pad(x[:, 128:], ((0, 0), (0, 256 - l)))

    dims = np.float32(l * EMBED_DIM)
    mag = np.float32(NOISE_ALPHA) / np.sqrt(dims)

    gathered = _sc_gather(table, x1, x2, b, l)
    g_t = jnp.transpose(gathered.reshape(b, l, EMBED_DIM), (1, 2, 0))
    out_t = _tc_noise_add(g_t, mag, b, l)
    return jnp.transpose(out_t, (2, 0, 1))
